# Initial kernel scaffold; baseline (speedup 1.0000x reference)
import jax
import jax.numpy as jnp
from jax.experimental import pallas as pl


def kernel(voxel_features, coors):
    raise NotImplementedError("write your pallas kernel here")


# speedup vs baseline: 1.2571x; 1.2571x over previous
import functools

import jax
import jax.numpy as jnp
from jax import lax
from jax.experimental import pallas as pl
from jax.experimental.pallas import tpu as pltpu
from jax.experimental.pallas import tpu_sc as plsc

_NX = 512
_NY = 512
_C = 64
_N = 30000
_P = _NX * _NY
_NP = 32768
_VPT = 1024
_HVPT = 512
_TR = _P + 16
_SEG = 16512
_OCC = 16 * _SEG

_mesh = plsc.VectorSubcoreMesh(core_axis_name="c", subcore_axis_name="s")


@functools.partial(
    pl.kernel,
    mesh=_mesh,
    out_type=[
        jax.ShapeDtypeStruct((_TR, 128), jnp.float32),
        jax.ShapeDtypeStruct((_OCC,), jnp.float32),
        jax.ShapeDtypeStruct((_OCC,), jnp.float32),
    ],
    scratch_types=[
        pltpu.VMEM((_HVPT, 128), jnp.float32),
        pltpu.VMEM((_VPT,), jnp.int32),
        pltpu.VMEM((_VPT,), jnp.int32),
        pltpu.VMEM((8, 128), jnp.int32),
        pltpu.VMEM((128,), jnp.float32),
        pltpu.VMEM((_SEG,), jnp.float32),
        pltpu.VMEM_SHARED((_OCC,), jnp.float32),
        pltpu.SemaphoreType.DMA,
        pltpu.SemaphoreType.DMA,
    ],
)
def _scatter_sc(vf_hbm, yc_hbm, xc_hbm, tcv_hbm, occ0_hbm, occ1_hbm,
                vf_v, y_v, x_v, idx_v, ones_v, zseg_v, occ_sh, sem, sem_occ):
    c = lax.axis_index("c")
    s = lax.axis_index("s")
    wid = c * 16 + s
    v0 = wid * _VPT

    def _fill_zero(i, carry):
        zseg_v[pl.ds(i * 16, 16)] = jnp.zeros((16,), jnp.float32)
        return carry

    lax.fori_loop(0, _SEG // 16, _fill_zero, 0)
    for i in range(8):
        ones_v[pl.ds(i * 16, 16)] = jnp.full((16,), 1.0, jnp.float32)

    pltpu.sync_copy(yc_hbm.at[pl.ds(v0, _VPT)], y_v)
    pltpu.sync_copy(xc_hbm.at[pl.ds(v0, _VPT)], x_v)

    for j in range(_VPT // 16):
        y = y_v[pl.ds(j * 16, 16)]
        x = x_v[pl.ds(j * 16, 16)]
        idx_v[j // 8, pl.ds((j % 8) * 16, 16)] = y * _NX + x

    pltpu.sync_copy(zseg_v, occ_sh.at[pl.ds(s * _SEG, _SEG)])
    plsc.subcore_barrier()

    occ_descs = [
        pltpu.async_copy(ones_v, occ_sh.at[idx_v.at[j]], sem_occ)
        for j in range(8)
    ]

    for h in range(2):
        pltpu.sync_copy(vf_hbm.at[pl.ds(v0 + h * _HVPT, _HVPT), :], vf_v)
        descs = [
            pltpu.async_copy(vf_v.at[pl.ds(j * 128, 128), :],
                             tcv_hbm.at[idx_v.at[4 * h + j]], sem)
            for j in range(4)
        ]
        for d in descs:
            d.wait()

    for d in occ_descs:
        d.wait()
    plsc.subcore_barrier()

    pltpu.sync_copy(occ_sh.at[pl.ds(s * _SEG, _SEG)], zseg_v)

    @pl.when(c == 0)
    def _exp0():
        pltpu.sync_copy(zseg_v, occ0_hbm.at[pl.ds(s * _SEG, _SEG)])

    @pl.when(c == 1)
    def _exp1():
        pltpu.sync_copy(zseg_v, occ1_hbm.at[pl.ds(s * _SEG, _SEG)])


def _tc_body(t_ref, o0_ref, o1_ref, out_ref):
    m = (o0_ref[...] + o1_ref[...]) != 0.0
    out_ref[...] = jnp.where(m, t_ref[:, :_C].T, 0.0)


def _transpose_tc(tcv, occ0, occ1):
    blk = 2048
    return pl.pallas_call(
        _tc_body,
        grid=(_P // blk,),
        in_specs=[
            pl.BlockSpec((blk, 128), lambda i: (i, 0)),
            pl.BlockSpec((1, blk), lambda i: (0, i)),
            pl.BlockSpec((1, blk), lambda i: (0, i)),
        ],
        out_specs=pl.BlockSpec((_C, blk), lambda i: (0, i)),
        out_shape=jax.ShapeDtypeStruct((_C, _P), jnp.float32),
        compiler_params=pltpu.CompilerParams(
            dimension_semantics=("parallel",)),
    )(tcv, occ0, occ1)


def kernel(voxel_features, coors):
    padn = _NP - _N
    vf_p = jnp.pad(voxel_features, ((0, padn), (0, 128 - _C)))
    yc = jnp.pad(coors[:, 1].astype(jnp.int32), (0, padn),
                 constant_values=_NY)
    xc = jnp.pad(coors[:, 2].astype(jnp.int32), (0, padn))
    tcv, occ0, occ1 = _scatter_sc(vf_p, yc, xc)
    canvas = _transpose_tc(tcv, occ0.reshape(1, -1), occ1.reshape(1, -1))
    return canvas.reshape(1, _C, _NY, _NX)
